# baseline (speedup 1.0000x reference)
import jax
import jax.numpy as jnp
from jax.experimental import pallas as pl
from jax.experimental.pallas import tpu as pltpu

L = 1024
KNN = 30
BF = jnp.bfloat16
F32 = jnp.float32


def _body(xf_ref, W1_ref, b1_ref, g1_ref, be1_ref, rm1_ref, rv1_ref,
          W2_ref, b2_ref, g2_ref, be2_ref, rm2_ref, rv2_ref, out_ref,
          vals_ref, nbr_ref):
    ft = xf_ref[0]
    sq = jnp.sum(ft * ft, axis=1, keepdims=True)
    sq_row = jnp.reshape(jnp.sum(ft * ft, axis=1), (1, L))
    G = jax.lax.dot_general(ft, ft, (((1,), (1,)), ((), ())))
    vals_ref[...] = sq + sq_row - 2.0 * G

    cols = jax.lax.broadcasted_iota(jnp.int32, (1, L), 1)
    tcols = jax.lax.broadcasted_iota(jnp.int32, (1, 32), 1)

    RB = 8

    def extract_block(bi, _):
        blk0 = vals_ref[pl.ds(bi * RB, RB), :]

        def step(t, carry):
            bv, nb = carry
            m = jnp.min(bv, axis=1, keepdims=True)
            idxm = jnp.min(jnp.where(bv == m, cols, L), axis=1,
                           keepdims=True)
            bv = jnp.where(cols == idxm, jnp.inf, bv)
            return bv, jnp.where(tcols == t, idxm, nb)

        _, nb = jax.lax.fori_loop(
            0, KNN, step, (blk0, jnp.zeros((RB, 32), jnp.int32)),
            unroll=2)
        nbr_ref[pl.ds(bi * RB, RB), :] = nb
        return 0

    jax.lax.fori_loop(0, L // RB, extract_block, 0)
    nbr = nbr_ref[...]

    cols16 = jax.lax.broadcasted_iota(jnp.int16, (1, L), 1)
    nbr16 = nbr.astype(jnp.int16)
    one_bf = jnp.ones((), BF)
    zero_bf = jnp.zeros((), BF)
    Mb = jnp.zeros((L, L), BF)
    for t in range(KNN):
        Mb = Mb + jnp.where(cols16 == nbr16[:, t:t + 1], one_bf, zero_bf)
    ones_col = jnp.ones((L, 1), BF)
    deg = jax.lax.dot_general(Mb, ones_col, (((0,), (0,)), ((), ())),
                              preferred_element_type=F32)
    degc = jnp.maximum(deg, 1.0)

    def bn(h, g_r, be_r, rm_r, rv_r):
        return (h - rm_r[0]) / jnp.sqrt(rv_r[0] + 1e-5) * g_r[0] + be_r[0]

    def mdot(h, dims):
        h_hi = h.astype(BF)
        h_lo = (h - h_hi.astype(F32)).astype(BF)
        return (jax.lax.dot_general(Mb, h_hi, dims, preferred_element_type=F32)
                + jax.lax.dot_general(Mb, h_lo, dims, preferred_element_type=F32))

    def v2v(h):
        E = mdot(h, (((1,), (0,)), ((), ()))) * (1.0 / KNN)
        Vsum = mdot(E, (((0,), (0,)), ((), ())))
        return Vsum / degc

    h = jax.lax.dot_general(ft, W1_ref[...], (((1,), (0,)), ((), ())))
    h = bn(h + b1_ref[0], g1_ref, be1_ref, rm1_ref, rv1_ref)
    h = jax.nn.relu(v2v(h))
    h = jax.lax.dot_general(h, W2_ref[...], (((1,), (0,)), ((), ())))
    h = bn(h + b2_ref[0], g2_ref, be2_ref, rm2_ref, rv2_ref)
    out_ref[0] = v2v(h)


def kernel(x, W1, b1, g1, be1, rm1, rv1, W2, b2, g2, be2, rm2, rv2):
    B, C, H, W = x.shape
    xf = x.reshape(B, L, C)
    vec = lambda v: v.reshape(1, -1)
    full = lambda r: pl.BlockSpec((1, r.shape[1]), lambda i: (0, 0))

    out = pl.pallas_call(
        _body,
        grid=(B,),
        in_specs=[
            pl.BlockSpec((1, L, C), lambda i: (i, 0, 0)),
            pl.BlockSpec(W1.shape, lambda i: (0, 0)),
            full(vec(b1)), full(vec(g1)), full(vec(be1)),
            full(vec(rm1)), full(vec(rv1)),
            pl.BlockSpec(W2.shape, lambda i: (0, 0)),
            full(vec(b2)), full(vec(g2)), full(vec(be2)),
            full(vec(rm2)), full(vec(rv2)),
        ],
        out_specs=pl.BlockSpec((1, L, W2.shape[1]), lambda i: (i, 0, 0)),
        out_shape=jax.ShapeDtypeStruct((B, L, W2.shape[1]), jnp.float32),
        scratch_shapes=[pltpu.VMEM((L, L), jnp.float32),
                        pltpu.VMEM((L, 32), jnp.int32)],
    )(xf, W1, vec(b1), vec(g1), vec(be1), vec(rm1), vec(rv1),
      W2, vec(b2), vec(g2), vec(be2), vec(rm2), vec(rv2))

    return out.reshape(B, -1, H, W)


# speedup vs baseline: 10.0552x; 10.0552x over previous
import jax
import jax.numpy as jnp
from jax.experimental import pallas as pl
from jax.experimental.pallas import tpu as pltpu

L = 1024
KNN = 30
BF = jnp.bfloat16
F32 = jnp.float32


def _body(xf_ref, W1_ref, b1_ref, g1_ref, be1_ref, rm1_ref, rv1_ref,
          W2_ref, b2_ref, g2_ref, be2_ref, rm2_ref, rv2_ref, out_ref,
          vals_ref, nbr_ref):
    ft = xf_ref[0]
    sq = jnp.sum(ft * ft, axis=1, keepdims=True)
    sq_row = jnp.reshape(jnp.sum(ft * ft, axis=1), (1, L))
    G = jax.lax.dot_general(ft, ft, (((1,), (1,)), ((), ())))
    vals_ref[...] = sq + sq_row - 2.0 * G

    cols = jax.lax.broadcasted_iota(jnp.int32, (1, L), 1)
    tcols = jax.lax.broadcasted_iota(jnp.int32, (1, 32), 1)

    def step(t, carry):
        mprev, cprev, nbr = carry
        vals = vals_ref[...]
        valid = (vals > mprev) | ((vals == mprev) & (cols > cprev))
        m = jnp.min(jnp.where(valid, vals, jnp.inf), axis=1, keepdims=True)
        rowflag = m > mprev
        pick = (vals == m) & (rowflag | (cols > cprev))
        idxm = jnp.min(jnp.where(pick, cols, L), axis=1, keepdims=True)
        return m, idxm, jnp.where(tcols == t, idxm, nbr)

    _, _, nbr = jax.lax.fori_loop(
        0, KNN, step,
        (jnp.full((L, 1), -jnp.inf, jnp.float32),
         jnp.full((L, 1), -1, jnp.int32),
         jnp.zeros((L, 32), jnp.int32)), unroll=2)

    cols16 = jax.lax.broadcasted_iota(jnp.int16, (1, L), 1)
    nbr16 = nbr.astype(jnp.int16)
    one_bf = jnp.ones((), BF)
    zero_bf = jnp.zeros((), BF)
    Mb = jnp.zeros((L, L), BF)
    for t in range(KNN):
        Mb = Mb + jnp.where(cols16 == nbr16[:, t:t + 1], one_bf, zero_bf)
    ones_col = jnp.ones((L, 1), BF)
    deg = jax.lax.dot_general(Mb, ones_col, (((0,), (0,)), ((), ())),
                              preferred_element_type=F32)
    degc = jnp.maximum(deg, 1.0)

    def bn(h, g_r, be_r, rm_r, rv_r):
        return (h - rm_r[0]) / jnp.sqrt(rv_r[0] + 1e-5) * g_r[0] + be_r[0]

    def mdot(h, dims):
        h_hi = h.astype(BF)
        h_lo = (h - h_hi.astype(F32)).astype(BF)
        return (jax.lax.dot_general(Mb, h_hi, dims, preferred_element_type=F32)
                + jax.lax.dot_general(Mb, h_lo, dims, preferred_element_type=F32))

    def v2v(h):
        E = mdot(h, (((1,), (0,)), ((), ()))) * (1.0 / KNN)
        Vsum = mdot(E, (((0,), (0,)), ((), ())))
        return Vsum / degc

    h = jax.lax.dot_general(ft, W1_ref[...], (((1,), (0,)), ((), ())))
    h = bn(h + b1_ref[0], g1_ref, be1_ref, rm1_ref, rv1_ref)
    h = jax.nn.relu(v2v(h))
    h = jax.lax.dot_general(h, W2_ref[...], (((1,), (0,)), ((), ())))
    h = bn(h + b2_ref[0], g2_ref, be2_ref, rm2_ref, rv2_ref)
    out_ref[0] = v2v(h)


def kernel(x, W1, b1, g1, be1, rm1, rv1, W2, b2, g2, be2, rm2, rv2):
    B, C, H, W = x.shape
    xf = x.reshape(B, L, C)
    vec = lambda v: v.reshape(1, -1)
    full = lambda r: pl.BlockSpec((1, r.shape[1]), lambda i: (0, 0))

    out = pl.pallas_call(
        _body,
        grid=(B,),
        in_specs=[
            pl.BlockSpec((1, L, C), lambda i: (i, 0, 0)),
            pl.BlockSpec(W1.shape, lambda i: (0, 0)),
            full(vec(b1)), full(vec(g1)), full(vec(be1)),
            full(vec(rm1)), full(vec(rv1)),
            pl.BlockSpec(W2.shape, lambda i: (0, 0)),
            full(vec(b2)), full(vec(g2)), full(vec(be2)),
            full(vec(rm2)), full(vec(rv2)),
        ],
        out_specs=pl.BlockSpec((1, L, W2.shape[1]), lambda i: (i, 0, 0)),
        out_shape=jax.ShapeDtypeStruct((B, L, W2.shape[1]), jnp.float32),
        scratch_shapes=[pltpu.VMEM((L, L), jnp.float32),
                        pltpu.VMEM((L, 32), jnp.int32)],
    )(xf, W1, vec(b1), vec(g1), vec(be1), vec(rm1), vec(rv1),
      W2, vec(b2), vec(g2), vec(be2), vec(rm2), vec(rv2))

    return out.reshape(B, -1, H, W)


# speedup vs baseline: 15.9069x; 1.5820x over previous
import jax
import jax.numpy as jnp
from jax.experimental import pallas as pl
from jax.experimental.pallas import tpu as pltpu

L = 1024
KNN = 30
BF = jnp.bfloat16
F32 = jnp.float32


def _body(xf_ref, W1_ref, b1_ref, g1_ref, be1_ref, rm1_ref, rv1_ref,
          W2_ref, b2_ref, g2_ref, be2_ref, rm2_ref, rv2_ref, out_ref,
          vals_ref):
    ft = xf_ref[0]
    sq = jnp.sum(ft * ft, axis=1, keepdims=True)
    sq_row = jnp.reshape(jnp.sum(ft * ft, axis=1), (1, L))
    G = jax.lax.dot_general(ft, ft, (((1,), (1,)), ((), ())))
    vals_ref[...] = sq + sq_row - 2.0 * G

    cols = jax.lax.broadcasted_iota(jnp.int32, (1, L), 1)
    tcols = jax.lax.broadcasted_iota(jnp.int32, (1, 32), 1)

    def step(t, nbr):
        vals = vals_ref[...]
        m = jnp.min(vals, axis=1, keepdims=True)
        eq = vals == m
        idxm = jnp.min(jnp.where(eq, cols, L), axis=1, keepdims=True)
        vals_ref[...] = jnp.where(cols == idxm, jnp.inf, vals)
        return jnp.where(tcols == t, idxm, nbr)

    nbr = jax.lax.fori_loop(
        0, KNN, step, jnp.zeros((L, 32), jnp.int32), unroll=2)

    cols16 = jax.lax.broadcasted_iota(jnp.int16, (1, L), 1)
    nbr16 = nbr.astype(jnp.int16)
    one_bf = jnp.ones((), BF)
    zero_bf = jnp.zeros((), BF)
    Mb = jnp.zeros((L, L), BF)
    for t in range(KNN):
        Mb = Mb + jnp.where(cols16 == nbr16[:, t:t + 1], one_bf, zero_bf)
    ones_col = jnp.ones((L, 1), BF)
    deg = jax.lax.dot_general(Mb, ones_col, (((0,), (0,)), ((), ())),
                              preferred_element_type=F32)
    degc = jnp.maximum(deg, 1.0)

    def bn(h, g_r, be_r, rm_r, rv_r):
        return (h - rm_r[0]) / jnp.sqrt(rv_r[0] + 1e-5) * g_r[0] + be_r[0]

    def mdot(h, dims):
        h_hi = h.astype(BF)
        h_lo = (h - h_hi.astype(F32)).astype(BF)
        return (jax.lax.dot_general(Mb, h_hi, dims, preferred_element_type=F32)
                + jax.lax.dot_general(Mb, h_lo, dims, preferred_element_type=F32))

    def v2v(h):
        E = mdot(h, (((1,), (0,)), ((), ()))) * (1.0 / KNN)
        Vsum = mdot(E, (((0,), (0,)), ((), ())))
        return Vsum / degc

    h = jax.lax.dot_general(ft, W1_ref[...], (((1,), (0,)), ((), ())))
    h = bn(h + b1_ref[0], g1_ref, be1_ref, rm1_ref, rv1_ref)
    h = jax.nn.relu(v2v(h))
    h = jax.lax.dot_general(h, W2_ref[...], (((1,), (0,)), ((), ())))
    h = bn(h + b2_ref[0], g2_ref, be2_ref, rm2_ref, rv2_ref)
    out_ref[0] = v2v(h)


def kernel(x, W1, b1, g1, be1, rm1, rv1, W2, b2, g2, be2, rm2, rv2):
    B, C, H, W = x.shape
    xf = x.reshape(B, L, C)
    vec = lambda v: v.reshape(1, -1)
    full = lambda r: pl.BlockSpec((1, r.shape[1]), lambda i: (0, 0))

    out = pl.pallas_call(
        _body,
        grid=(B,),
        in_specs=[
            pl.BlockSpec((1, L, C), lambda i: (i, 0, 0)),
            pl.BlockSpec(W1.shape, lambda i: (0, 0)),
            full(vec(b1)), full(vec(g1)), full(vec(be1)),
            full(vec(rm1)), full(vec(rv1)),
            pl.BlockSpec(W2.shape, lambda i: (0, 0)),
            full(vec(b2)), full(vec(g2)), full(vec(be2)),
            full(vec(rm2)), full(vec(rv2)),
        ],
        out_specs=pl.BlockSpec((1, L, W2.shape[1]), lambda i: (i, 0, 0)),
        out_shape=jax.ShapeDtypeStruct((B, L, W2.shape[1]), jnp.float32),
        scratch_shapes=[pltpu.VMEM((L, L), jnp.float32)],
    )(xf, W1, vec(b1), vec(g1), vec(be1), vec(rm1), vec(rv1),
      W2, vec(b2), vec(g2), vec(be2), vec(rm2), vec(rv2))

    return out.reshape(B, -1, H, W)
